# baseline (speedup 1.0000x reference)
import jax
import jax.numpy as jnp
from jax.experimental import pallas as pl
from jax.experimental.pallas import tpu as pltpu

ROWS = 128
N = 100000
W = 2048
NBLK = (N + W - 1) // W
KPER = W // 128
TINY = 1e-20
NEG_INF = float("-inf")


def _body(l_ref, u_ref, out_ref, accv, acci):
    pid = pl.program_id(0)

    @pl.when(pid == 0)
    def _init():
        accv[...] = jnp.full((ROWS, 128), NEG_INF, jnp.float32)
        acci[...] = jnp.zeros((ROWS, 128), jnp.int32)

    u = u_ref[...]
    l = l_ref[...]
    t = TINY - jnp.log(u + TINY)
    x = l - jnp.log(t)
    col = jax.lax.broadcasted_iota(jnp.int32, (ROWS, W), 1) + pid * W
    x = jnp.where(col < N, x, NEG_INF)

    av = accv[...]
    ai = acci[...]
    base_chunk = pid * KPER
    for k in range(KPER):
        xk = x[:, k * 128:(k + 1) * 128]
        gt = xk > av
        av = jnp.where(gt, xk, av)
        ai = jnp.where(gt, jnp.int32(base_chunk + k), ai)
    accv[...] = av
    acci[...] = ai

    @pl.when(pid == NBLK - 1)
    def _fin():
        fv = accv[...]
        fi = acci[...]
        m = jnp.max(fv, axis=1, keepdims=True)
        lane = jax.lax.broadcasted_iota(jnp.int32, (ROWS, 128), 1)
        cand = jnp.where(fv == m, fi * 128 + lane, jnp.int32(2**30))
        out_ref[0, :] = jnp.min(cand, axis=1)


def kernel(logits, gumbel_u):
    out = pl.pallas_call(
        _body,
        grid=(NBLK,),
        in_specs=[
            pl.BlockSpec((ROWS, W), lambda j: (0, j)),
            pl.BlockSpec((ROWS, W), lambda j: (0, j)),
        ],
        out_specs=pl.BlockSpec((1, 128), lambda j: (0, 0)),
        out_shape=jax.ShapeDtypeStruct((1, 128), jnp.int32),
        scratch_shapes=[
            pltpu.VMEM((ROWS, 128), jnp.float32),
            pltpu.VMEM((ROWS, 128), jnp.int32),
        ],
    )(logits, gumbel_u)
    return out.reshape(ROWS)


# speedup vs baseline: 1.0584x; 1.0584x over previous
import jax
import jax.numpy as jnp
from jax.experimental import pallas as pl
from jax.experimental.pallas import tpu as pltpu

ROWS = 128
N = 100000
W = 8192
NBLK = (N + W - 1) // W
KPER = W // 128
TINY = 1e-20
NEG_INF = float("-inf")


def _body(l_ref, u_ref, out_ref, accv, acci):
    pid = pl.program_id(0)

    @pl.when(pid == 0)
    def _init():
        accv[...] = jnp.full((ROWS, 128), NEG_INF, jnp.float32)
        acci[...] = jnp.zeros((ROWS, 128), jnp.int32)

    u = u_ref[...]
    l = l_ref[...]
    t = TINY - jnp.log(u + TINY)
    x = l - jnp.log(t)
    col = jax.lax.broadcasted_iota(jnp.int32, (ROWS, W), 1) + pid * W
    x = jnp.where(col < N, x, NEG_INF)

    av = accv[...]
    ai = acci[...]
    base_chunk = pid * KPER
    for k in range(KPER):
        xk = x[:, k * 128:(k + 1) * 128]
        gt = xk > av
        av = jnp.where(gt, xk, av)
        ai = jnp.where(gt, jnp.int32(base_chunk + k), ai)
    accv[...] = av
    acci[...] = ai

    @pl.when(pid == NBLK - 1)
    def _fin():
        fv = accv[...]
        fi = acci[...]
        m = jnp.max(fv, axis=1, keepdims=True)
        lane = jax.lax.broadcasted_iota(jnp.int32, (ROWS, 128), 1)
        cand = jnp.where(fv == m, fi * 128 + lane, jnp.int32(2**30))
        out_ref[0, :] = jnp.min(cand, axis=1)


def kernel(logits, gumbel_u):
    out = pl.pallas_call(
        _body,
        grid=(NBLK,),
        in_specs=[
            pl.BlockSpec((ROWS, W), lambda j: (0, j)),
            pl.BlockSpec((ROWS, W), lambda j: (0, j)),
        ],
        out_specs=pl.BlockSpec((1, 128), lambda j: (0, 0)),
        out_shape=jax.ShapeDtypeStruct((1, 128), jnp.int32),
        scratch_shapes=[
            pltpu.VMEM((ROWS, 128), jnp.float32),
            pltpu.VMEM((ROWS, 128), jnp.int32),
        ],
    )(logits, gumbel_u)
    return out.reshape(ROWS)


# speedup vs baseline: 2.6574x; 2.5108x over previous
import jax
import jax.numpy as jnp
from jax.experimental import pallas as pl
from jax.experimental.pallas import tpu as pltpu

ROWS = 128
N = 100000
S = 2000
NBLK = N // S
KPER = S // 8
NACC = 8
TINY = 1e-20
NEG_INF = float("-inf")


def _body(l_ref, u_ref, out_ref, accv, acci):
    pid = pl.program_id(0)

    @pl.when(pid == 0)
    def _init():
        accv[...] = jnp.full((NACC * 8, ROWS), NEG_INF, jnp.float32)
        acci[...] = jnp.zeros((NACC * 8, ROWS), jnp.int32)

    av = [accv[a * 8:(a + 1) * 8, :] for a in range(NACC)]
    ai = [acci[a * 8:(a + 1) * 8, :] for a in range(NACC)]
    base = pid * KPER
    for k in range(KPER):
        a = k % NACC
        u = u_ref[k * 8:(k + 1) * 8, :]
        l = l_ref[k * 8:(k + 1) * 8, :]
        t = TINY - jnp.log(u + TINY)
        x = l - jnp.log(t)
        gt = x > av[a]
        av[a] = jnp.where(gt, x, av[a])
        ai[a] = jnp.where(gt, jnp.int32(base + k), ai[a])
    for a in range(NACC):
        accv[a * 8:(a + 1) * 8, :] = av[a]
        acci[a * 8:(a + 1) * 8, :] = ai[a]

    @pl.when(pid == NBLK - 1)
    def _fin():
        sub = jax.lax.broadcasted_iota(jnp.int32, (8, ROWS), 0)
        m = av[0]
        for a in range(1, NACC):
            m = jnp.maximum(m, av[a])
        m = jnp.max(m, axis=0, keepdims=True)
        best = jnp.full((8, ROWS), jnp.int32(2**30))
        for a in range(NACC):
            cand = jnp.where(av[a] == m, ai[a] * 8 + sub, jnp.int32(2**30))
            best = jnp.minimum(best, cand)
        out_ref[0, :] = jnp.min(best, axis=0)


def kernel(logits, gumbel_u):
    out = pl.pallas_call(
        _body,
        grid=(NBLK,),
        in_specs=[
            pl.BlockSpec((S, ROWS), lambda j: (j, 0)),
            pl.BlockSpec((S, ROWS), lambda j: (j, 0)),
        ],
        out_specs=pl.BlockSpec((1, ROWS), lambda j: (0, 0)),
        out_shape=jax.ShapeDtypeStruct((1, ROWS), jnp.int32),
        scratch_shapes=[
            pltpu.VMEM((NACC * 8, ROWS), jnp.float32),
            pltpu.VMEM((NACC * 8, ROWS), jnp.int32),
        ],
    )(logits.T, gumbel_u.T)
    return out.reshape(ROWS)


# speedup vs baseline: 3.7178x; 1.3990x over previous
import jax
import jax.numpy as jnp
from jax.experimental import pallas as pl
from jax.experimental.pallas import tpu as pltpu

ROWS = 128
N = 100000
S = 5000
NBLK = N // S
KPER = S // 8
NACC = 8
TINY = 1e-20
NEG_INF = float("-inf")


def _body(l_ref, u_ref, out_ref, accv, acci):
    pid = pl.program_id(0)

    @pl.when(pid == 0)
    def _init():
        accv[...] = jnp.full((NACC * 8, ROWS), NEG_INF, jnp.float32)
        acci[...] = jnp.zeros((NACC * 8, ROWS), jnp.int32)

    av = [accv[a * 8:(a + 1) * 8, :] for a in range(NACC)]
    ai = [acci[a * 8:(a + 1) * 8, :] for a in range(NACC)]
    base = pid * KPER
    for k in range(KPER):
        a = k % NACC
        u = u_ref[k * 8:(k + 1) * 8, :]
        l = l_ref[k * 8:(k + 1) * 8, :]
        t = TINY - jnp.log(u + TINY)
        x = l - jnp.log(t)
        gt = x > av[a]
        av[a] = jnp.where(gt, x, av[a])
        ai[a] = jnp.where(gt, jnp.int32(base + k), ai[a])
    for a in range(NACC):
        accv[a * 8:(a + 1) * 8, :] = av[a]
        acci[a * 8:(a + 1) * 8, :] = ai[a]

    @pl.when(pid == NBLK - 1)
    def _fin():
        sub = jax.lax.broadcasted_iota(jnp.int32, (8, ROWS), 0)
        m = av[0]
        for a in range(1, NACC):
            m = jnp.maximum(m, av[a])
        m = jnp.max(m, axis=0, keepdims=True)
        best = jnp.full((8, ROWS), jnp.int32(2**30))
        for a in range(NACC):
            cand = jnp.where(av[a] == m, ai[a] * 8 + sub, jnp.int32(2**30))
            best = jnp.minimum(best, cand)
        out_ref[0, :] = jnp.min(best, axis=0)


def kernel(logits, gumbel_u):
    out = pl.pallas_call(
        _body,
        grid=(NBLK,),
        in_specs=[
            pl.BlockSpec((S, ROWS), lambda j: (j, 0)),
            pl.BlockSpec((S, ROWS), lambda j: (j, 0)),
        ],
        out_specs=pl.BlockSpec((1, ROWS), lambda j: (0, 0)),
        out_shape=jax.ShapeDtypeStruct((1, ROWS), jnp.int32),
        scratch_shapes=[
            pltpu.VMEM((NACC * 8, ROWS), jnp.float32),
            pltpu.VMEM((NACC * 8, ROWS), jnp.int32),
        ],
    )(logits.T, gumbel_u.T)
    return out.reshape(ROWS)


# speedup vs baseline: 4.1340x; 1.1119x over previous
import jax
import jax.numpy as jnp
from jax.experimental import pallas as pl
from jax.experimental.pallas import tpu as pltpu

ROWS = 128
N = 100000
S = 10000
NBLK = N // S
KPER = S // 8
NACC = 8
TINY = 1e-20
NEG_INF = float("-inf")


def _body(l_ref, u_ref, out_ref, accv, acci):
    pid = pl.program_id(0)

    @pl.when(pid == 0)
    def _init():
        accv[...] = jnp.full((NACC * 8, ROWS), NEG_INF, jnp.float32)
        acci[...] = jnp.zeros((NACC * 8, ROWS), jnp.int32)

    av = [accv[a * 8:(a + 1) * 8, :] for a in range(NACC)]
    ai = [acci[a * 8:(a + 1) * 8, :] for a in range(NACC)]
    base = pid * KPER
    for k in range(KPER):
        a = k % NACC
        u = u_ref[k * 8:(k + 1) * 8, :]
        l = l_ref[k * 8:(k + 1) * 8, :]
        t = TINY - jnp.log(u + TINY)
        x = l - jnp.log(t)
        gt = x > av[a]
        av[a] = jnp.where(gt, x, av[a])
        ai[a] = jnp.where(gt, jnp.int32(base + k), ai[a])
    for a in range(NACC):
        accv[a * 8:(a + 1) * 8, :] = av[a]
        acci[a * 8:(a + 1) * 8, :] = ai[a]

    @pl.when(pid == NBLK - 1)
    def _fin():
        sub = jax.lax.broadcasted_iota(jnp.int32, (8, ROWS), 0)
        m = av[0]
        for a in range(1, NACC):
            m = jnp.maximum(m, av[a])
        m = jnp.max(m, axis=0, keepdims=True)
        best = jnp.full((8, ROWS), jnp.int32(2**30))
        for a in range(NACC):
            cand = jnp.where(av[a] == m, ai[a] * 8 + sub, jnp.int32(2**30))
            best = jnp.minimum(best, cand)
        out_ref[0, :] = jnp.min(best, axis=0)


def kernel(logits, gumbel_u):
    out = pl.pallas_call(
        _body,
        grid=(NBLK,),
        in_specs=[
            pl.BlockSpec((S, ROWS), lambda j: (j, 0)),
            pl.BlockSpec((S, ROWS), lambda j: (j, 0)),
        ],
        out_specs=pl.BlockSpec((1, ROWS), lambda j: (0, 0)),
        out_shape=jax.ShapeDtypeStruct((1, ROWS), jnp.int32),
        scratch_shapes=[
            pltpu.VMEM((NACC * 8, ROWS), jnp.float32),
            pltpu.VMEM((NACC * 8, ROWS), jnp.int32),
        ],
    )(logits.T, gumbel_u.T)
    return out.reshape(ROWS)
